# Initial kernel scaffold; baseline (speedup 1.0000x reference)
import jax
import jax.numpy as jnp
from jax.experimental import pallas as pl


def kernel(x, edge_index, W1, b1, W2, b2, Wu, bu, We, be):
    raise NotImplementedError("write your pallas kernel here")


# speedup vs baseline: 2.9249x; 2.9249x over previous
import functools

import jax
import jax.numpy as jnp
from jax import lax
from jax.experimental import pallas as pl
from jax.experimental.pallas import tpu as pltpu
from jax.experimental.pallas import tpu_sc as plsc

N_NODES = 10000
N_EDGES = 320000
D_IN = 128
D_HID = 128
C_CLS = 5

GROUP = 128
N_WORKERS = 32
G_MAIN = 80
N_GROUPS = N_WORKERS * G_MAIN
E_PAD = N_GROUPS * GROUP

PW = 32
ROW_BLK = 1000


def _tc_body(x_ref, w1_ref, b1_ref, w2_ref, b2_ref, wc_ref, bc_ref, out_ref):
    h = jnp.maximum(jnp.dot(x_ref[...], w1_ref[...],
                            preferred_element_type=jnp.float32) + b1_ref[...], 0.0)
    h = jnp.maximum(jnp.dot(h, w2_ref[...],
                            preferred_element_type=jnp.float32) + b2_ref[...], 0.0)
    out_ref[...] = jnp.dot(h, wc_ref[...],
                           preferred_element_type=jnp.float32) + bc_ref[...]


def _tc_heads(x, W1, b1, W2, b2, Wc, bc):
    grid = (N_NODES // ROW_BLK,)
    return pl.pallas_call(
        _tc_body,
        grid=grid,
        in_specs=[
            pl.BlockSpec((ROW_BLK, D_IN), lambda i: (i, 0)),
            pl.BlockSpec((D_IN, D_HID), lambda i: (0, 0)),
            pl.BlockSpec((1, D_HID), lambda i: (0, 0)),
            pl.BlockSpec((D_HID, D_HID), lambda i: (0, 0)),
            pl.BlockSpec((1, D_HID), lambda i: (0, 0)),
            pl.BlockSpec((D_HID, 128), lambda i: (0, 0)),
            pl.BlockSpec((1, 128), lambda i: (0, 0)),
        ],
        out_specs=pl.BlockSpec((ROW_BLK, 128), lambda i: (i, 0)),
        out_shape=jax.ShapeDtypeStruct((N_NODES, 128), jnp.float32),
    )(x, W1, b1.reshape(1, -1), W2, b2.reshape(1, -1), Wc, bc.reshape(1, -1))


def _sc_edge_body(p_hbm, q_hbm, src_hbm, dst_hbm, out_hbm,
                  idx_s, idx_d, rows_p, rows_q, sem_p, sem_q):
    wid = lax.axis_index("s") * 2 + lax.axis_index("c")
    g0 = wid * G_MAIN

    pltpu.sync_copy(src_hbm.at[pl.ds(g0, G_MAIN)], idx_s.at[pl.ds(0, G_MAIN)])
    pltpu.sync_copy(dst_hbm.at[pl.ds(g0, G_MAIN)], idx_d.at[pl.ds(0, G_MAIN)])

    def do_group(g, js):
        cp = pltpu.async_copy(p_hbm.at[idx_s.at[js]], rows_p, sem_p)
        cq = pltpu.async_copy(q_hbm.at[idx_d.at[js]], rows_q, sem_q)
        cp.wait()
        cq.wait()

        def add_row(i, _):
            rows_p[i, pl.ds(0, 16)] = rows_p[i, pl.ds(0, 16)] + rows_q[i, pl.ds(0, 16)]
            rows_p[i, pl.ds(16, 16)] = rows_p[i, pl.ds(16, 16)] + rows_q[i, pl.ds(16, 16)]
            return 0

        lax.fori_loop(0, GROUP, add_row, 0, unroll=4)
        pltpu.sync_copy(rows_p, out_hbm.at[pl.ds(g * GROUP, GROUP)])

    def main_group(j, _):
        do_group(g0 + j, j)
        return 0

    lax.fori_loop(0, G_MAIN, main_group, 0)


def _sc_edge_pot(p32, q32, src2d, dst2d):
    mesh = plsc.VectorSubcoreMesh(core_axis_name="c", subcore_axis_name="s")
    f = pl.kernel(
        _sc_edge_body,
        out_type=jax.ShapeDtypeStruct((E_PAD, PW), jnp.float32),
        mesh=mesh,
        scratch_types=[
            pltpu.VMEM((G_MAIN, GROUP), jnp.int32),
            pltpu.VMEM((G_MAIN, GROUP), jnp.int32),
            pltpu.VMEM((GROUP, PW), jnp.float32),
            pltpu.VMEM((GROUP, PW), jnp.float32),
            pltpu.SemaphoreType.DMA,
            pltpu.SemaphoreType.DMA,
        ],
        compiler_params=pltpu.CompilerParams(use_tc_tiling_on_sc=False),
    )
    return f(p32, q32, src2d, dst2d)


@jax.jit
def kernel(x, edge_index, W1, b1, W2, b2, Wu, bu, We, be):
    zcol = jnp.zeros((D_HID, 27), jnp.float32)
    Wc = jnp.concatenate([
        Wu,
        zcol,
        We[:D_HID],
        jnp.zeros((D_HID, 7), jnp.float32),
        We[D_HID:],
        jnp.zeros((D_HID, 39), jnp.float32),
    ], axis=1)
    bc = jnp.concatenate([
        bu, jnp.zeros((27,), jnp.float32),
        jnp.zeros((25,), jnp.float32), jnp.zeros((7,), jnp.float32),
        be, jnp.zeros((39,), jnp.float32),
    ])

    comb = _tc_heads(x, W1, b1, W2, b2, Wc, bc)
    unary = comb[:, :C_CLS]
    p32 = comb[:, 32:64]
    q32 = comb[:, 64:96]

    pad = jnp.zeros((2, E_PAD - N_EDGES), jnp.int32)
    ei = jnp.concatenate([edge_index, pad], axis=1)
    src2d = ei[0].reshape(N_GROUPS, GROUP)
    dst2d = ei[1].reshape(N_GROUPS, GROUP)
    out32 = _sc_edge_pot(p32, q32, src2d, dst2d)
    return (unary, out32[:N_EDGES, :C_CLS * C_CLS])


# speedup vs baseline: 4.5934x; 1.5705x over previous
import functools

import jax
import jax.numpy as jnp
from jax import lax
from jax.experimental import pallas as pl
from jax.experimental.pallas import tpu as pltpu
from jax.experimental.pallas import tpu_sc as plsc

N_NODES = 10000
N_EDGES = 320000
D_IN = 128
D_HID = 128
C_CLS = 5
CC = C_CLS * C_CLS

GROUP = 128
N_WORKERS = 32
G_MAIN = 80
N_GROUPS = N_WORKERS * G_MAIN
E_PAD = N_GROUPS * GROUP
N_REAL_GROUPS = N_EDGES // GROUP

PW = 32
STAGE_W = GROUP * CC
ROW_BLK = 1000


def _tc_body(x_ref, w1_ref, b1_ref, w2_ref, b2_ref, wc_ref, bc_ref, out_ref):
    h = jnp.maximum(jnp.dot(x_ref[...], w1_ref[...],
                            preferred_element_type=jnp.float32) + b1_ref[...], 0.0)
    h = jnp.maximum(jnp.dot(h, w2_ref[...],
                            preferred_element_type=jnp.float32) + b2_ref[...], 0.0)
    out_ref[...] = jnp.dot(h, wc_ref[...],
                           preferred_element_type=jnp.float32) + bc_ref[...]


def _tc_heads(x, W1, b1, W2, b2, Wc, bc):
    grid = (N_NODES // ROW_BLK,)
    return pl.pallas_call(
        _tc_body,
        grid=grid,
        in_specs=[
            pl.BlockSpec((ROW_BLK, D_IN), lambda i: (i, 0)),
            pl.BlockSpec((D_IN, D_HID), lambda i: (0, 0)),
            pl.BlockSpec((1, D_HID), lambda i: (0, 0)),
            pl.BlockSpec((D_HID, D_HID), lambda i: (0, 0)),
            pl.BlockSpec((1, D_HID), lambda i: (0, 0)),
            pl.BlockSpec((D_HID, 128), lambda i: (0, 0)),
            pl.BlockSpec((1, 128), lambda i: (0, 0)),
        ],
        out_specs=pl.BlockSpec((ROW_BLK, 128), lambda i: (i, 0)),
        out_shape=jax.ShapeDtypeStruct((N_NODES, 128), jnp.float32),
    )(x, W1, b1.reshape(1, -1), W2, b2.reshape(1, -1), Wc, bc.reshape(1, -1))


def _sc_edge_body(p_hbm, q_hbm, src_hbm, dst_hbm, out_hbm,
                  idx_s, idx_d, rp0, rp1, rq0, rq1, st0, st1,
                  sp0, sp1, sq0, sq1, so0, so1):
    RP, RQ, ST = [rp0, rp1], [rq0, rq1], [st0, st1]
    SP, SQ, SO = [sp0, sp1], [sq0, sq1], [so0, so1]

    wid = lax.axis_index("s") * 2 + lax.axis_index("c")
    g0 = wid * G_MAIN

    pltpu.sync_copy(src_hbm.at[pl.ds(g0, G_MAIN)], idx_s)
    pltpu.sync_copy(dst_hbm.at[pl.ds(g0, G_MAIN)], idx_d)

    def is_real(j):
        return jnp.logical_and(j < G_MAIN, g0 + j < N_REAL_GROUPS)

    def fire(j, b):
        @pl.when(is_real(j))
        def _():
            pltpu.async_copy(p_hbm.at[idx_s.at[j]], RP[b], SP[b])
            pltpu.async_copy(q_hbm.at[idx_d.at[j]], RQ[b], SQ[b])

    def process(j, b):
        @pl.when(is_real(j))
        def _():
            pltpu.make_async_copy(p_hbm.at[idx_s.at[j]], RP[b], SP[b]).wait()
            pltpu.make_async_copy(q_hbm.at[idx_d.at[j]], RQ[b], SQ[b]).wait()

            @pl.when(j >= 2)
            def _():
                pltpu.make_async_copy(ST[b].at[pl.ds(0, STAGE_W)],
                                      out_hbm.at[pl.ds(0, STAGE_W)],
                                      SO[b]).wait()

            def add_row(i, _):
                a0 = RP[b][i, pl.ds(0, 16)] + RQ[b][i, pl.ds(0, 16)]
                a1 = RP[b][i, pl.ds(16, 16)] + RQ[b][i, pl.ds(16, 16)]
                ST[b][pl.ds(i * CC, 16)] = a0
                ST[b][pl.ds(i * CC + 16, 16)] = a1
                return 0

            lax.fori_loop(0, GROUP, add_row, 0, unroll=4)
            pltpu.async_copy(ST[b].at[pl.ds(0, STAGE_W)],
                             out_hbm.at[pl.ds((g0 + j) * STAGE_W, STAGE_W)],
                             SO[b])

    fire(0, 0)
    fire(1, 1)

    def outer(t, _):
        j0 = t * 2
        for b in range(2):
            process(j0 + b, b)
            fire(j0 + b + 2, b)
        return 0

    lax.fori_loop(0, G_MAIN // 2, outer, 0)

    for b in range(2):
        pltpu.make_async_copy(ST[b].at[pl.ds(0, STAGE_W)],
                              out_hbm.at[pl.ds(0, STAGE_W)], SO[b]).wait()


def _sc_edge_pot(p32, q32, src2d, dst2d):
    mesh = plsc.VectorSubcoreMesh(core_axis_name="c", subcore_axis_name="s")
    f = pl.kernel(
        _sc_edge_body,
        out_type=jax.ShapeDtypeStruct((N_EDGES * CC,), jnp.float32),
        mesh=mesh,
        scratch_types=[
            pltpu.VMEM((G_MAIN, GROUP), jnp.int32),
            pltpu.VMEM((G_MAIN, GROUP), jnp.int32),
            pltpu.VMEM((GROUP, PW), jnp.float32),
            pltpu.VMEM((GROUP, PW), jnp.float32),
            pltpu.VMEM((GROUP, PW), jnp.float32),
            pltpu.VMEM((GROUP, PW), jnp.float32),
            pltpu.VMEM((STAGE_W + 16,), jnp.float32),
            pltpu.VMEM((STAGE_W + 16,), jnp.float32),
            pltpu.SemaphoreType.DMA,
            pltpu.SemaphoreType.DMA,
            pltpu.SemaphoreType.DMA,
            pltpu.SemaphoreType.DMA,
            pltpu.SemaphoreType.DMA,
            pltpu.SemaphoreType.DMA,
        ],
        compiler_params=pltpu.CompilerParams(use_tc_tiling_on_sc=False),
    )
    return f(p32, q32, src2d, dst2d)


@jax.jit
def kernel(x, edge_index, W1, b1, W2, b2, Wu, bu, We, be):
    zcol = jnp.zeros((D_HID, 27), jnp.float32)
    Wc = jnp.concatenate([
        Wu,
        zcol,
        We[:D_HID],
        jnp.zeros((D_HID, 7), jnp.float32),
        We[D_HID:],
        jnp.zeros((D_HID, 39), jnp.float32),
    ], axis=1)
    bc = jnp.concatenate([
        bu, jnp.zeros((27,), jnp.float32),
        jnp.zeros((25,), jnp.float32), jnp.zeros((7,), jnp.float32),
        be, jnp.zeros((39,), jnp.float32),
    ])

    comb = _tc_heads(x, W1, b1, W2, b2, Wc, bc)
    unary = comb[:, :C_CLS]
    p32 = comb[:, 32:64]
    q32 = comb[:, 64:96]

    pad = jnp.zeros((2, E_PAD - N_EDGES), jnp.int32)
    ei = jnp.concatenate([edge_index, pad], axis=1)
    src2d = ei[0].reshape(N_GROUPS, GROUP)
    dst2d = ei[1].reshape(N_GROUPS, GROUP)
    out_flat = _sc_edge_pot(p32, q32, src2d, dst2d)
    return (unary, out_flat.reshape(N_EDGES, CC))


# speedup vs baseline: 5.7716x; 1.2565x over previous
import functools

import jax
import jax.numpy as jnp
from jax import lax
from jax.experimental import pallas as pl
from jax.experimental.pallas import tpu as pltpu
from jax.experimental.pallas import tpu_sc as plsc

N_NODES = 10000
N_EDGES = 320000
D_IN = 128
D_HID = 128
C_CLS = 5
CC = C_CLS * C_CLS

GROUP = 128
N_WORKERS = 32
G_MAIN = 80
N_REAL_GROUPS = N_EDGES // GROUP

PW = 32
ROW_BLK = 1000


def _tc_body(x_ref, w1_ref, b1_ref, w2_ref, b2_ref, wc_ref, bc_ref, out_ref):
    h = jnp.maximum(jnp.dot(x_ref[...], w1_ref[...],
                            preferred_element_type=jnp.float32) + b1_ref[...], 0.0)
    h = jnp.maximum(jnp.dot(h, w2_ref[...],
                            preferred_element_type=jnp.float32) + b2_ref[...], 0.0)
    out_ref[...] = jnp.dot(h, wc_ref[...],
                           preferred_element_type=jnp.float32) + bc_ref[...]


def _tc_heads(x, W1, b1, W2, b2, Wc, bc):
    grid = (N_NODES // ROW_BLK,)
    return pl.pallas_call(
        _tc_body,
        grid=grid,
        in_specs=[
            pl.BlockSpec((ROW_BLK, D_IN), lambda i: (i, 0)),
            pl.BlockSpec((D_IN, D_HID), lambda i: (0, 0)),
            pl.BlockSpec((1, D_HID), lambda i: (0, 0)),
            pl.BlockSpec((D_HID, D_HID), lambda i: (0, 0)),
            pl.BlockSpec((1, D_HID), lambda i: (0, 0)),
            pl.BlockSpec((D_HID, 128), lambda i: (0, 0)),
            pl.BlockSpec((1, 128), lambda i: (0, 0)),
        ],
        out_specs=pl.BlockSpec((ROW_BLK, 128), lambda i: (i, 0)),
        out_shape=jax.ShapeDtypeStruct((N_NODES, 128), jnp.float32),
    )(x, W1, b1.reshape(1, -1), W2, b2.reshape(1, -1), Wc, bc.reshape(1, -1))


def _sc_edge_body(p_hbm, q_hbm, src_hbm, dst_hbm, out_hbm,
                  idx_s, idx_d, rp0, rp1, rq0, rq1, st0, st1,
                  sp0, sp1, sq0, sq1, so0, so1):
    RP, RQ, ST = [rp0, rp1], [rq0, rq1], [st0, st1]
    SP, SQ, SO = [sp0, sp1], [sq0, sq1], [so0, so1]

    wid = lax.axis_index("s") * 2 + lax.axis_index("c")
    g0 = wid * G_MAIN
    gl = jnp.minimum(g0, N_REAL_GROUPS - G_MAIN)
    off = g0 - gl

    pltpu.sync_copy(src_hbm.at[pl.ds(gl, G_MAIN)], idx_s)
    pltpu.sync_copy(dst_hbm.at[pl.ds(gl, G_MAIN)], idx_d)

    def is_real(j):
        return jnp.logical_and(j < G_MAIN, g0 + j < N_REAL_GROUPS)

    def fire(j, b):
        @pl.when(is_real(j))
        def _():
            pltpu.async_copy(p_hbm.at[idx_s.at[j + off]], RP[b], SP[b])
            pltpu.async_copy(q_hbm.at[idx_d.at[j + off]], RQ[b], SQ[b])

    def process(j, b):
        @pl.when(is_real(j))
        def _():
            pltpu.make_async_copy(p_hbm.at[idx_s.at[j + off]], RP[b], SP[b]).wait()
            pltpu.make_async_copy(q_hbm.at[idx_d.at[j + off]], RQ[b], SQ[b]).wait()

            @pl.when(j >= 2)
            def _():
                pltpu.make_async_copy(ST[b], out_hbm.at[pl.ds(0, GROUP)],
                                      SO[b]).wait()

            def add_row(i, _):
                a0 = RP[b][i, pl.ds(0, 16)] + RQ[b][i, pl.ds(0, 16)]
                a1 = RP[b][i, pl.ds(9, 16)] + RQ[b][i, pl.ds(9, 16)]
                ST[b][i, pl.ds(0, 16)] = a0
                ST[b][i, pl.ds(9, 16)] = a1
                return 0

            lax.fori_loop(0, GROUP, add_row, 0, unroll=4)
            pltpu.async_copy(ST[b], out_hbm.at[pl.ds((g0 + j) * GROUP, GROUP)],
                             SO[b])

    fire(0, 0)
    fire(1, 1)

    def outer(t, _):
        j0 = t * 2
        for b in range(2):
            process(j0 + b, b)
            fire(j0 + b + 2, b)
        return 0

    lax.fori_loop(0, G_MAIN // 2, outer, 0)

    for b in range(2):
        pltpu.make_async_copy(ST[b], out_hbm.at[pl.ds(0, GROUP)], SO[b]).wait()


def _sc_edge_pot(p32, q32, src2d, dst2d):
    mesh = plsc.VectorSubcoreMesh(core_axis_name="c", subcore_axis_name="s")
    f = pl.kernel(
        _sc_edge_body,
        out_type=jax.ShapeDtypeStruct((N_EDGES, CC), jnp.float32),
        mesh=mesh,
        scratch_types=[
            pltpu.VMEM((G_MAIN, GROUP), jnp.int32),
            pltpu.VMEM((G_MAIN, GROUP), jnp.int32),
            pltpu.VMEM((GROUP, PW), jnp.float32),
            pltpu.VMEM((GROUP, PW), jnp.float32),
            pltpu.VMEM((GROUP, PW), jnp.float32),
            pltpu.VMEM((GROUP, PW), jnp.float32),
            pltpu.VMEM((GROUP, CC), jnp.float32),
            pltpu.VMEM((GROUP, CC), jnp.float32),
            pltpu.SemaphoreType.DMA,
            pltpu.SemaphoreType.DMA,
            pltpu.SemaphoreType.DMA,
            pltpu.SemaphoreType.DMA,
            pltpu.SemaphoreType.DMA,
            pltpu.SemaphoreType.DMA,
        ],
        compiler_params=pltpu.CompilerParams(use_tc_tiling_on_sc=False),
    )
    return f(p32, q32, src2d, dst2d)


@jax.jit
def kernel(x, edge_index, W1, b1, W2, b2, Wu, bu, We, be):
    zcol = jnp.zeros((D_HID, 27), jnp.float32)
    Wc = jnp.concatenate([
        Wu,
        zcol,
        We[:D_HID],
        jnp.zeros((D_HID, 7), jnp.float32),
        We[D_HID:],
        jnp.zeros((D_HID, 39), jnp.float32),
    ], axis=1)
    bc = jnp.concatenate([
        bu, jnp.zeros((27,), jnp.float32),
        jnp.zeros((25,), jnp.float32), jnp.zeros((7,), jnp.float32),
        be, jnp.zeros((39,), jnp.float32),
    ])

    comb = _tc_heads(x, W1, b1, W2, b2, Wc, bc)
    unary = comb[:, :C_CLS]
    p32 = comb[:, 32:64]
    q32 = comb[:, 64:96]

    src2d = edge_index[0].reshape(N_REAL_GROUPS, GROUP)
    dst2d = edge_index[1].reshape(N_REAL_GROUPS, GROUP)
    edge_pot = _sc_edge_pot(p32, q32, src2d, dst2d)
    return (unary, edge_pot)


# speedup vs baseline: 5.7776x; 1.0010x over previous
import functools

import jax
import jax.numpy as jnp
from jax import lax
from jax.experimental import layout as jlayout
from jax.experimental import pallas as pl
from jax.experimental.pallas import tpu as pltpu
from jax.experimental.pallas import tpu_sc as plsc

N_NODES = 10000
N_EDGES = 320000
D_IN = 128
D_HID = 128
C_CLS = 5
CC = C_CLS * C_CLS

GROUP = 128
N_WORKERS = 32
G_MAIN = 80
N_REAL_GROUPS = N_EDGES // GROUP

PW = 32
ROW_BLK = 1000


def _tc_body(x_ref, w1_ref, b1_ref, w2_ref, b2_ref, wc_ref, bc_ref, out_ref):
    h = jnp.maximum(jnp.dot(x_ref[...], w1_ref[...],
                            preferred_element_type=jnp.float32) + b1_ref[...], 0.0)
    h = jnp.maximum(jnp.dot(h, w2_ref[...],
                            preferred_element_type=jnp.float32) + b2_ref[...], 0.0)
    out_ref[...] = jnp.dot(h, wc_ref[...],
                           preferred_element_type=jnp.float32) + bc_ref[...]


def _tc_heads(x, W1, b1, W2, b2, Wc, bc):
    grid = (N_NODES // ROW_BLK,)
    return pl.pallas_call(
        _tc_body,
        grid=grid,
        in_specs=[
            pl.BlockSpec((ROW_BLK, D_IN), lambda i: (i, 0)),
            pl.BlockSpec((D_IN, D_HID), lambda i: (0, 0)),
            pl.BlockSpec((1, D_HID), lambda i: (0, 0)),
            pl.BlockSpec((D_HID, D_HID), lambda i: (0, 0)),
            pl.BlockSpec((1, D_HID), lambda i: (0, 0)),
            pl.BlockSpec((D_HID, 128), lambda i: (0, 0)),
            pl.BlockSpec((1, 128), lambda i: (0, 0)),
        ],
        out_specs=pl.BlockSpec((ROW_BLK, 128), lambda i: (i, 0)),
        out_shape=jax.ShapeDtypeStruct((N_NODES, 128), jnp.float32),
    )(x, W1, b1.reshape(1, -1), W2, b2.reshape(1, -1), Wc, bc.reshape(1, -1))


def _sc_edge_body(p_hbm, q_hbm, src_hbm, dst_hbm, out_hbm,
                  idx_s, idx_d, rp0, rp1, rq0, rq1, st0, st1,
                  sp0, sp1, sq0, sq1, so0, so1):
    RP, RQ, ST = [rp0, rp1], [rq0, rq1], [st0, st1]
    SP, SQ, SO = [sp0, sp1], [sq0, sq1], [so0, so1]

    wid = lax.axis_index("s") * 2 + lax.axis_index("c")
    g0 = wid * G_MAIN
    gl = jnp.minimum(g0, N_REAL_GROUPS - G_MAIN)
    off = g0 - gl

    pltpu.sync_copy(src_hbm.at[pl.ds(gl, G_MAIN)], idx_s)
    pltpu.sync_copy(dst_hbm.at[pl.ds(gl, G_MAIN)], idx_d)

    def is_real(j):
        return jnp.logical_and(j < G_MAIN, g0 + j < N_REAL_GROUPS)

    def fire(j, b):
        @pl.when(is_real(j))
        def _():
            pltpu.async_copy(p_hbm.at[idx_s.at[j + off]], RP[b], SP[b])
            pltpu.async_copy(q_hbm.at[idx_d.at[j + off]], RQ[b], SQ[b])

    def process(j, b):
        @pl.when(is_real(j))
        def _():
            pltpu.make_async_copy(p_hbm.at[idx_s.at[j + off]], RP[b], SP[b]).wait()
            pltpu.make_async_copy(q_hbm.at[idx_d.at[j + off]], RQ[b], SQ[b]).wait()

            @pl.when(j >= 2)
            def _():
                pltpu.make_async_copy(ST[b], out_hbm.at[pl.ds(0, GROUP)],
                                      SO[b]).wait()

            def add_row(i, _):
                a0 = RP[b][i, pl.ds(0, 16)] + RQ[b][i, pl.ds(0, 16)]
                a1 = RP[b][i, pl.ds(9, 16)] + RQ[b][i, pl.ds(9, 16)]
                ST[b][i, pl.ds(0, 16)] = a0
                ST[b][i, pl.ds(9, 16)] = a1
                return 0

            lax.fori_loop(0, GROUP, add_row, 0, unroll=4)
            pltpu.async_copy(ST[b], out_hbm.at[pl.ds((g0 + j) * GROUP, GROUP)],
                             SO[b])

    fire(0, 0)
    fire(1, 1)

    def outer(t, _):
        j0 = t * 2
        for b in range(2):
            process(j0 + b, b)
            fire(j0 + b + 2, b)
        return 0

    lax.fori_loop(0, G_MAIN // 2, outer, 0)

    for b in range(2):
        pltpu.make_async_copy(ST[b], out_hbm.at[pl.ds(0, GROUP)], SO[b]).wait()


def _sc_edge_pot(p32, q32, src2d, dst2d):
    mesh = plsc.VectorSubcoreMesh(core_axis_name="c", subcore_axis_name="s")
    f = pl.kernel(
        _sc_edge_body,
        out_type=jax.ShapeDtypeStruct((N_EDGES, CC), jnp.float32),
        mesh=mesh,
        scratch_types=[
            pltpu.VMEM((G_MAIN, GROUP), jnp.int32),
            pltpu.VMEM((G_MAIN, GROUP), jnp.int32),
            pltpu.VMEM((GROUP, PW), jnp.float32),
            pltpu.VMEM((GROUP, PW), jnp.float32),
            pltpu.VMEM((GROUP, PW), jnp.float32),
            pltpu.VMEM((GROUP, PW), jnp.float32),
            pltpu.VMEM((GROUP, CC), jnp.float32),
            pltpu.VMEM((GROUP, CC), jnp.float32),
            pltpu.SemaphoreType.DMA,
            pltpu.SemaphoreType.DMA,
            pltpu.SemaphoreType.DMA,
            pltpu.SemaphoreType.DMA,
            pltpu.SemaphoreType.DMA,
            pltpu.SemaphoreType.DMA,
        ],
        compiler_params=pltpu.CompilerParams(use_tc_tiling_on_sc=False),
    )
    return f(p32, q32, src2d, dst2d)


def _kernel_impl(x, edge_index, W1, b1, W2, b2, Wu, bu, We, be):
    zcol = jnp.zeros((D_HID, 27), jnp.float32)
    Wc = jnp.concatenate([
        Wu,
        zcol,
        We[:D_HID],
        jnp.zeros((D_HID, 7), jnp.float32),
        We[D_HID:],
        jnp.zeros((D_HID, 39), jnp.float32),
    ], axis=1)
    bc = jnp.concatenate([
        bu, jnp.zeros((27,), jnp.float32),
        jnp.zeros((25,), jnp.float32), jnp.zeros((7,), jnp.float32),
        be, jnp.zeros((39,), jnp.float32),
    ])

    comb = _tc_heads(x, W1, b1, W2, b2, Wc, bc)
    unary = comb[:, :C_CLS]
    p32 = comb[:, 32:64]
    q32 = comb[:, 64:96]

    src2d = edge_index[0].reshape(N_REAL_GROUPS, GROUP)
    dst2d = edge_index[1].reshape(N_REAL_GROUPS, GROUP)
    edge_pot = _sc_edge_pot(p32, q32, src2d, dst2d)
    return (unary, edge_pot)


@functools.lru_cache(maxsize=None)
def _jit_for(sharding):
    return jax.jit(
        _kernel_impl,
        out_shardings=(
            jlayout.Format(),
            jlayout.Format(jlayout.Layout(major_to_minor=(0, 1), tiling=()),
                           sharding),
        ),
    )


def kernel(x, *args):
    sharding = getattr(x, "sharding", None)
    if sharding is None:
        sharding = jax.sharding.SingleDeviceSharding(jax.devices()[0])
    return _jit_for(sharding)(x, *args)
